# Initial kernel scaffold; baseline (speedup 1.0000x reference)
import jax
import jax.numpy as jnp
from jax.experimental import pallas as pl


def kernel(x, node_types, W1, b1, W2, b2):
    raise NotImplementedError("write your pallas kernel here")


# speedup vs baseline: 1.5714x; 1.5714x over previous
import jax
import jax.numpy as jnp
from jax.experimental import pallas as pl

N = 100000
D = 128
H = 128
O = 128
T = 4
R = 2000


def _body(nt_ref, x_ref, w1_ref, b1_ref, w2_ref, b2_ref, o_ref):
    xb = x_ref[...].astype(jnp.bfloat16)
    h = jnp.dot(xb, w1_ref[...], preferred_element_type=jnp.float32)
    h = jnp.maximum(h + b1_ref[...], 0.0)
    t = nt_ref[0]
    grp = jax.lax.broadcasted_iota(jnp.int32, (1, T * H), 1) // H
    h = jnp.where(grp == t, h, 0.0).astype(jnp.bfloat16)
    o = jnp.dot(h, w2_ref[...], preferred_element_type=jnp.float32)
    b2 = b2_ref[...]
    for tt in range(T):
        o = o + jnp.where(t == tt, b2[tt][None, :], 0.0)
    o_ref[...] = o


def kernel(x, node_types, W1, b1, W2, b2):
    nb = N // R
    w1cat = jnp.transpose(W1, (0, 2, 1)).reshape(T, D, H)
    w1cat = jnp.concatenate([w1cat[t] for t in range(T)], axis=1)
    w2stk = jnp.transpose(W2, (0, 2, 1)).reshape(T * H, O)
    b1cat = b1.reshape(1, T * H)
    nt3 = node_types.reshape(nb, R, 1)
    grid_spec = pl.GridSpec(
        grid=(nb,),
        in_specs=[
            pl.BlockSpec((1, R, 1), lambda i: (i, 0, 0)),
            pl.BlockSpec((R, D), lambda i: (i, 0)),
            pl.BlockSpec((D, T * H), lambda i: (0, 0)),
            pl.BlockSpec((1, T * H), lambda i: (0, 0)),
            pl.BlockSpec((T * H, O), lambda i: (0, 0)),
            pl.BlockSpec((T, O), lambda i: (0, 0)),
        ],
        out_specs=pl.BlockSpec((R, O), lambda i: (i, 0)),
    )
    return pl.pallas_call(
        _body,
        grid_spec=grid_spec,
        out_shape=jax.ShapeDtypeStruct((N, O), jnp.float32),
    )(nt3, x, w1cat.astype(jnp.bfloat16), b1cat, w2stk.astype(jnp.bfloat16), b2)
